# baseline (speedup 1.0000x reference)
import functools

import jax
import jax.numpy as jnp
from jax import lax
from jax.experimental import pallas as pl
from jax.experimental.pallas import tpu as pltpu
from jax.experimental.pallas import tpu_sc as plsc

N = 10000
D = 128
NC = 2
NS = 16
NW = NC * NS
BLK = 128
N_ACC = ((N + 1 + NS * 8 - 1) // (NS * 8)) * (NS * 8)
RPT = N_ACC // NS

_MESH = plsc.VectorSubcoreMesh(core_axis_name="c", subcore_axis_name="s")


def _make_agg(e_pad: int):
    epw = e_pad // NW
    nblk = epw // BLK

    @functools.partial(
        pl.kernel,
        out_type=jax.ShapeDtypeStruct((NC, N_ACC, D), jnp.float32),
        mesh=_MESH,
        scratch_types=[
            pltpu.VMEM((epw // BLK, BLK), jnp.int32),
            pltpu.VMEM((BLK,), jnp.int32),
            pltpu.VMEM((BLK,), jnp.int32),
            pltpu.VMEM((BLK, D), jnp.float32),
            pltpu.VMEM((BLK, D), jnp.float32),
            pltpu.VMEM_SHARED((N_ACC, D), jnp.float32),
            pltpu.SemaphoreType.DMA,
            pltpu.SemaphoreType.DMA,
            pltpu.SemaphoreType.DMA,
        ],
    )
    def agg(x_hbm, src_hbm, dst_hbm, zf_hbm, agg_out,
            src_all, dst_v0, dst_v1, rows_v0, rows_v1, agg_sp,
            sem_i, sem_g0, sem_g1):
        c = lax.axis_index("c")
        s = lax.axis_index("s")
        wid = c * NS + s
        base = wid * epw

        del dst_v1, rows_v1, sem_i, sem_g1

        pltpu.sync_copy(src_hbm.at[wid], src_all)
        pltpu.sync_copy(zf_hbm.at[pl.ds(s * RPT, RPT)],
                        agg_sp.at[pl.ds(s * RPT, RPT)])
        plsc.subcore_barrier()

        def step(t, carry):
            pltpu.async_copy(x_hbm.at[src_all.at[t]], rows_v0, sem_g0)
            pltpu.sync_copy(dst_hbm.at[pl.ds(base + t * BLK, BLK)], dst_v0)
            pltpu.make_async_copy(x_hbm.at[src_all.at[t]],
                                  rows_v0, sem_g0).wait()
            pltpu.sync_copy(rows_v0, agg_sp.at[dst_v0], add=True)
            return carry

        lax.fori_loop(0, nblk, step, 0)
        plsc.subcore_barrier()

        pltpu.sync_copy(agg_sp.at[pl.ds(s * RPT, RPT)],
                        agg_out.at[c, pl.ds(s * RPT, RPT)])

    return agg


def _make_cnt(e_pad: int):
    epw = e_pad // NW
    nblk = epw // BLK

    @functools.partial(
        pl.kernel,
        out_type=jax.ShapeDtypeStruct((NC, N_ACC, D), jnp.float32),
        mesh=_MESH,
        scratch_types=[
            pltpu.VMEM((BLK,), jnp.int32),
            pltpu.VMEM((BLK,), jnp.int32),
            pltpu.VMEM((BLK, D), jnp.float32),
            pltpu.VMEM_SHARED((N_ACC, D), jnp.float32),
            pltpu.SemaphoreType.DMA,
        ],
    )
    def cnt(dst_hbm, zf_hbm, ones_hbm, cnt_out,
            dst_v0, dst_v1, ones_v, cnt_sp, sem_i):
        c = lax.axis_index("c")
        s = lax.axis_index("s")
        wid = c * NS + s
        base = wid * epw
        dst_v = (dst_v0, dst_v1)

        pltpu.sync_copy(zf_hbm.at[pl.ds(s * RPT, RPT)],
                        cnt_sp.at[pl.ds(s * RPT, RPT)])
        pltpu.sync_copy(ones_hbm, ones_v)
        plsc.subcore_barrier()

        pltpu.sync_copy(dst_hbm.at[pl.ds(base, BLK)], dst_v0)
        pltpu.async_copy(dst_hbm.at[pl.ds(base + BLK, BLK)], dst_v1, sem_i)

        def half(t, cur, nxt):
            pltpu.make_async_copy(dst_hbm.at[pl.ds(base, BLK)],
                                  dst_v[nxt], sem_i).wait()
            pltpu.sync_copy(ones_v, cnt_sp.at[dst_v[cur]], add=True)
            off2 = base + jnp.minimum(t + 2, nblk - 1) * BLK
            pltpu.async_copy(dst_hbm.at[pl.ds(off2, BLK)], dst_v[cur], sem_i)

        def macro(m, carry):
            half(2 * m, 0, 1)
            half(2 * m + 1, 1, 0)
            return carry

        lax.fori_loop(0, nblk // 2, macro, 0)
        pltpu.make_async_copy(dst_hbm.at[pl.ds(base, BLK)],
                              dst_v0, sem_i).wait()
        plsc.subcore_barrier()

        pltpu.sync_copy(cnt_sp.at[pl.ds(s * RPT, RPT)],
                        cnt_out.at[c, pl.ds(s * RPT, RPT)])

    return cnt


def _combine1_body(parts_ref, cnt_ref, x_ref, wl_ref, wr_ref, b_ref, out_ref):
    a = parts_ref[0] + parts_ref[1]
    cnt = cnt_ref[0, :, 0:1] + cnt_ref[1, :, 0:1]
    mean = a / jnp.maximum(cnt, 1.0)
    h = (jnp.dot(mean, wl_ref[...], preferred_element_type=jnp.float32)
         + jnp.dot(x_ref[...], wr_ref[...], preferred_element_type=jnp.float32)
         + b_ref[...])
    out_ref[...] = jnp.maximum(h, 0.0)


def _combine2_body(parts_ref, cnt_ref, h_ref, wl_ref, wr_ref, b_ref,
                   wm_ref, bm_ref, out_ref):
    a = parts_ref[0] + parts_ref[1]
    cnt = cnt_ref[0, :, 0:1] + cnt_ref[1, :, 0:1]
    mean = a / jnp.maximum(cnt, 1.0)
    t = (jnp.dot(mean, wl_ref[...], preferred_element_type=jnp.float32)
         + jnp.dot(h_ref[...], wr_ref[...], preferred_element_type=jnp.float32)
         + b_ref[...])
    out_ref[...] = (jnp.dot(t, wm_ref[...], preferred_element_type=jnp.float32)
                    + bm_ref[...])


_BN = 2000


def _tc_combine1(parts, cnt, x, wlT, wrT, b):
    grid = (N // _BN,)
    return pl.pallas_call(
        _combine1_body,
        grid=grid,
        in_specs=[
            pl.BlockSpec((2, _BN, D), lambda i: (0, i, 0)),
            pl.BlockSpec((2, _BN, D), lambda i: (0, i, 0)),
            pl.BlockSpec((_BN, D), lambda i: (i, 0)),
            pl.BlockSpec((D, D), lambda i: (0, 0)),
            pl.BlockSpec((D, D), lambda i: (0, 0)),
            pl.BlockSpec((1, D), lambda i: (0, 0)),
        ],
        out_specs=pl.BlockSpec((_BN, D), lambda i: (i, 0)),
        out_shape=jax.ShapeDtypeStruct((N, D), jnp.float32),
    )(parts, cnt, x, wlT, wrT, b)


def _tc_combine2(parts, cnt, h, wlT, wrT, b, wmT, bm):
    grid = (N // _BN,)
    return pl.pallas_call(
        _combine2_body,
        grid=grid,
        in_specs=[
            pl.BlockSpec((2, _BN, D), lambda i: (0, i, 0)),
            pl.BlockSpec((2, _BN, D), lambda i: (0, i, 0)),
            pl.BlockSpec((_BN, D), lambda i: (i, 0)),
            pl.BlockSpec((D, D), lambda i: (0, 0)),
            pl.BlockSpec((D, D), lambda i: (0, 0)),
            pl.BlockSpec((1, D), lambda i: (0, 0)),
            pl.BlockSpec((D, D), lambda i: (0, 0)),
            pl.BlockSpec((1, D), lambda i: (0, 0)),
        ],
        out_specs=pl.BlockSpec((_BN, D), lambda i: (i, 0)),
        out_shape=jax.ShapeDtypeStruct((N, D), jnp.float32),
    )(parts, cnt, h, wlT, wrT, b, wmT, bm)


def kernel(x, edge_index, Wl1, bl1, Wr1, Wl2, bl2, Wr2, Wm, bm):
    e = edge_index.shape[1]
    gran = NW * BLK * 2
    e_pad = ((e + gran - 1) // gran) * gran
    pad = e_pad - e
    src = jnp.concatenate(
        [edge_index[0].astype(jnp.int32), jnp.zeros((pad,), jnp.int32)])
    dst = jnp.concatenate(
        [edge_index[1].astype(jnp.int32), jnp.full((pad,), N, jnp.int32)])
    nblk = e_pad // (NW * BLK)
    src3d = src.reshape(NW, nblk, BLK)

    zf = jnp.zeros((N_ACC, D), jnp.float32)
    ones = jnp.ones((BLK, D), jnp.float32)

    agg_fn = _make_agg(e_pad)
    cnt_fn = _make_cnt(e_pad)

    cnt = cnt_fn(dst, zf, ones)
    parts1 = agg_fn(x, src3d, dst, zf)
    h = _tc_combine1(parts1, cnt, x, Wl1.T, Wr1.T, bl1.reshape(1, D))
    parts2 = agg_fn(h, src3d, dst, zf)
    q_m = _tc_combine2(parts2, cnt, h, Wl2.T, Wr2.T, bl2.reshape(1, D),
                       Wm.T, bm.reshape(1, D))
    return q_m


# speedup vs baseline: 1.3425x; 1.3425x over previous
import functools

import jax
import jax.numpy as jnp
from jax import lax
from jax.experimental import pallas as pl
from jax.experimental.pallas import tpu as pltpu
from jax.experimental.pallas import tpu_sc as plsc

N = 10000
D = 128
NC = 2
NS = 16
NW = NC * NS
BLK = 128
N_ACC = ((N + 1 + NS * 8 - 1) // (NS * 8)) * (NS * 8)
RPT = N_ACC // NS

_MESH = plsc.VectorSubcoreMesh(core_axis_name="c", subcore_axis_name="s")


def _make_agg(e_pad: int):
    epw = e_pad // NW
    nblk = epw // BLK

    @functools.partial(
        pl.kernel,
        out_type=jax.ShapeDtypeStruct((NC, N_ACC, D), jnp.float32),
        mesh=_MESH,
        scratch_types=[
            pltpu.VMEM((BLK,), jnp.int32),
            pltpu.VMEM((BLK,), jnp.int32),
            pltpu.VMEM((BLK, D), jnp.float32),
            pltpu.VMEM_SHARED((N_ACC, D), jnp.float32),
            pltpu.SemaphoreType.DMA,
        ],
    )
    def agg(x_hbm, src_hbm, dst_hbm, zf_hbm, agg_out,
            src_v, dst_v, rows_v, agg_sp, sem):
        c = lax.axis_index("c")
        s = lax.axis_index("s")
        wid = c * NS + s

        pltpu.sync_copy(zf_hbm.at[pl.ds(s * RPT, RPT)],
                        agg_sp.at[pl.ds(s * RPT, RPT)])
        plsc.subcore_barrier()

        base = wid * epw

        def step(t, carry):
            off = base + t * BLK
            pltpu.sync_copy(src_hbm.at[pl.ds(off, BLK)], src_v)
            pltpu.sync_copy(dst_hbm.at[pl.ds(off, BLK)], dst_v)
            pltpu.async_copy(x_hbm.at[src_v], rows_v, sem).wait()
            pltpu.sync_copy(rows_v, agg_sp.at[dst_v], add=True)
            return carry

        lax.fori_loop(0, nblk, step, 0)
        plsc.subcore_barrier()

        pltpu.sync_copy(agg_sp.at[pl.ds(s * RPT, RPT)],
                        agg_out.at[c, pl.ds(s * RPT, RPT)])

    return agg


def _make_cnt(e_pad: int):
    epw = e_pad // NW
    nblk = epw // BLK

    @functools.partial(
        pl.kernel,
        out_type=jax.ShapeDtypeStruct((NC, N_ACC, D), jnp.float32),
        mesh=_MESH,
        scratch_types=[
            pltpu.VMEM((BLK,), jnp.int32),
            pltpu.VMEM((BLK, D), jnp.float32),
            pltpu.VMEM_SHARED((N_ACC, D), jnp.float32),
        ],
    )
    def cnt(dst_hbm, zf_hbm, ones_hbm, cnt_out, dst_v, ones_v, cnt_sp):
        c = lax.axis_index("c")
        s = lax.axis_index("s")
        wid = c * NS + s

        pltpu.sync_copy(zf_hbm.at[pl.ds(s * RPT, RPT)],
                        cnt_sp.at[pl.ds(s * RPT, RPT)])
        pltpu.sync_copy(ones_hbm, ones_v)
        plsc.subcore_barrier()

        base = wid * epw

        def step(t, carry):
            off = base + t * BLK
            pltpu.sync_copy(dst_hbm.at[pl.ds(off, BLK)], dst_v)
            pltpu.sync_copy(ones_v, cnt_sp.at[dst_v], add=True)
            return carry

        lax.fori_loop(0, nblk, step, 0)
        plsc.subcore_barrier()

        pltpu.sync_copy(cnt_sp.at[pl.ds(s * RPT, RPT)],
                        cnt_out.at[c, pl.ds(s * RPT, RPT)])

    return cnt


def _combine1_body(parts_ref, cnt_ref, x_ref, wl_ref, wr_ref, b_ref, out_ref):
    a = parts_ref[0] + parts_ref[1]
    cnt = cnt_ref[0, :, 0:1] + cnt_ref[1, :, 0:1]
    mean = a / jnp.maximum(cnt, 1.0)
    h = (jnp.dot(mean, wl_ref[...], preferred_element_type=jnp.float32)
         + jnp.dot(x_ref[...], wr_ref[...], preferred_element_type=jnp.float32)
         + b_ref[...])
    out_ref[...] = jnp.maximum(h, 0.0)


def _combine2_body(parts_ref, cnt_ref, h_ref, wl_ref, wr_ref, b_ref,
                   wm_ref, bm_ref, out_ref):
    a = parts_ref[0] + parts_ref[1]
    cnt = cnt_ref[0, :, 0:1] + cnt_ref[1, :, 0:1]
    mean = a / jnp.maximum(cnt, 1.0)
    t = (jnp.dot(mean, wl_ref[...], preferred_element_type=jnp.float32)
         + jnp.dot(h_ref[...], wr_ref[...], preferred_element_type=jnp.float32)
         + b_ref[...])
    out_ref[...] = (jnp.dot(t, wm_ref[...], preferred_element_type=jnp.float32)
                    + bm_ref[...])


_BN = 2000


def _tc_combine1(parts, cnt, x, wlT, wrT, b):
    grid = (N // _BN,)
    return pl.pallas_call(
        _combine1_body,
        grid=grid,
        in_specs=[
            pl.BlockSpec((2, _BN, D), lambda i: (0, i, 0)),
            pl.BlockSpec((2, _BN, D), lambda i: (0, i, 0)),
            pl.BlockSpec((_BN, D), lambda i: (i, 0)),
            pl.BlockSpec((D, D), lambda i: (0, 0)),
            pl.BlockSpec((D, D), lambda i: (0, 0)),
            pl.BlockSpec((1, D), lambda i: (0, 0)),
        ],
        out_specs=pl.BlockSpec((_BN, D), lambda i: (i, 0)),
        out_shape=jax.ShapeDtypeStruct((N, D), jnp.float32),
    )(parts, cnt, x, wlT, wrT, b)


def _tc_combine2(parts, cnt, h, wlT, wrT, b, wmT, bm):
    grid = (N // _BN,)
    return pl.pallas_call(
        _combine2_body,
        grid=grid,
        in_specs=[
            pl.BlockSpec((2, _BN, D), lambda i: (0, i, 0)),
            pl.BlockSpec((2, _BN, D), lambda i: (0, i, 0)),
            pl.BlockSpec((_BN, D), lambda i: (i, 0)),
            pl.BlockSpec((D, D), lambda i: (0, 0)),
            pl.BlockSpec((D, D), lambda i: (0, 0)),
            pl.BlockSpec((1, D), lambda i: (0, 0)),
            pl.BlockSpec((D, D), lambda i: (0, 0)),
            pl.BlockSpec((1, D), lambda i: (0, 0)),
        ],
        out_specs=pl.BlockSpec((_BN, D), lambda i: (i, 0)),
        out_shape=jax.ShapeDtypeStruct((N, D), jnp.float32),
    )(parts, cnt, h, wlT, wrT, b, wmT, bm)


def kernel(x, edge_index, Wl1, bl1, Wr1, Wl2, bl2, Wr2, Wm, bm):
    e = edge_index.shape[1]
    gran = NW * BLK
    e_pad = ((e + gran - 1) // gran) * gran
    pad = e_pad - e
    src = jnp.concatenate(
        [edge_index[0].astype(jnp.int32), jnp.zeros((pad,), jnp.int32)])
    dst = jnp.concatenate(
        [edge_index[1].astype(jnp.int32), jnp.full((pad,), N, jnp.int32)])

    zf = jnp.zeros((N_ACC, D), jnp.float32)
    ones = jnp.ones((BLK, D), jnp.float32)

    agg_fn = _make_agg(e_pad)
    cnt_fn = _make_cnt(e_pad)

    cnt = cnt_fn(dst, zf, ones)
    parts1 = agg_fn(x, src, dst, zf)
    h = _tc_combine1(parts1, cnt, x, Wl1.T, Wr1.T, bl1.reshape(1, D))
    parts2 = agg_fn(h, src, dst, zf)
    q_m = _tc_combine2(parts2, cnt, h, Wl2.T, Wr2.T, bl2.reshape(1, D),
                       Wm.T, bm.reshape(1, D))
    return q_m
